# baseline (device time: 352155 ns/iter reference)
import jax
import jax.numpy as jnp
from jax import lax
from jax.experimental import pallas as pl
from jax.experimental.pallas import tpu as pltpu

N_DEV = 32
PAD = 31
BLK = PAD + 1


def kernel(x, dest):
    n_local, d_model = x.shape

    order = jnp.argsort(dest, stable=True)
    x_sorted = x[order]
    counts = jnp.bincount(dest, length=N_DEV)
    lo = jnp.cumsum(counts) - counts

    idx = jnp.clip(lo[:, None] + jnp.arange(PAD)[None, :], 0, n_local - 1)
    data_blocks = x_sorted[idx]
    hdr = jnp.broadcast_to(
        counts[:, None, None].astype(jnp.float32), (N_DEV, 1, d_model)
    )
    send_buf = jnp.concatenate([hdr, data_blocks], axis=1)

    def body(send_ref, recv_ref, send_sems, recv_sems):
        me = lax.axis_index("i")

        descs = []
        for d in range(N_DEV):
            descs.append(
                pltpu.make_async_remote_copy(
                    src_ref=send_ref.at[d],
                    dst_ref=recv_ref.at[me],
                    send_sem=send_sems.at[d],
                    recv_sem=recv_sems.at[me],
                    device_id=(d,),
                    device_id_type=pl.DeviceIdType.MESH,
                )
            )

        for d in range(N_DEV):
            pl.when(d != me)(descs[d].start)

        recv_ref[pl.ds(me, 1)] = send_ref[pl.ds(me, 1)]

        for s in range(N_DEV):
            wdesc = pltpu.make_async_remote_copy(
                src_ref=send_ref.at[s],
                dst_ref=recv_ref.at[s],
                send_sem=send_sems.at[s],
                recv_sem=recv_sems.at[s],
                device_id=(s,),
                device_id_type=pl.DeviceIdType.MESH,
            )
            pl.when(s != me)(wdesc.wait_recv)
        for d in range(N_DEV):
            pl.when(d != me)(descs[d].wait_send)

    recv = pl.pallas_call(
        body,
        out_shape=jax.ShapeDtypeStruct((N_DEV, BLK, d_model), jnp.float32),
        in_specs=[pl.BlockSpec(memory_space=pltpu.VMEM)],
        out_specs=pl.BlockSpec(memory_space=pltpu.VMEM),
        scratch_shapes=[
            pltpu.SemaphoreType.DMA((N_DEV,)),
            pltpu.SemaphoreType.DMA((N_DEV,)),
        ],
    )(send_buf)

    cin = recv[:, 0, 0].astype(jnp.int32)
    cum = jnp.cumsum(cin)
    k = jnp.arange(n_local)
    src = jnp.searchsorted(cum, k, side="right")
    pos = k - (cum[src] - cin[src])
    flat = recv[:, 1:, :].reshape(N_DEV * PAD, d_model)
    return flat[src * PAD + pos]


# device time: 36925 ns/iter; 9.5370x vs baseline; 9.5370x over previous
import jax
import jax.numpy as jnp
from jax import lax
from jax.experimental import pallas as pl
from jax.experimental.pallas import tpu as pltpu

N_DEV = 32
BLK = 32
PAD = BLK - 1
FLAT = N_DEV * BLK

f32 = jnp.float32


def _iota(shape, dim, dtype=jnp.int32):
    return lax.broadcasted_iota(dtype, shape, dim)


def kernel(x, dest):
    n_local, d_model = x.shape
    dest_row = dest.reshape(1, n_local)

    def body(x_ref, dest_ref, out_ref, send_ref, recv_ref, send_sems, recv_sems):
        me = lax.axis_index("i")
        x_val = x_ref[:, :]
        d_row = dest_ref[:, :]

        O2 = (_iota((N_DEV, n_local), 0) == d_row).astype(f32)
        U = (_iota((n_local, n_local), 0) <= _iota((n_local, n_local), 1)).astype(f32)
        C2 = jnp.dot(O2, U, preferred_element_type=f32, precision=lax.Precision.HIGHEST)
        rank_row = jnp.sum(O2 * C2, axis=0, keepdims=True) - 1.0
        counts_col = jnp.sum(O2, axis=1, keepdims=True)

        f_row = d_row.astype(f32) * BLK + 1.0 + rank_row
        St = (
            jnp.abs(_iota((FLAT, n_local), 0).astype(f32) - f_row) < 0.5
        ).astype(f32)
        hdr_sel = (
            _iota((FLAT, N_DEV), 0) == _iota((FLAT, N_DEV), 1) * BLK
        ).astype(f32)
        counts_bcast = jnp.broadcast_to(counts_col, (N_DEV, d_model))
        send_ref[:, :] = jnp.dot(St, x_val, preferred_element_type=f32, precision=lax.Precision.HIGHEST) + jnp.dot(
            hdr_sel, counts_bcast, preferred_element_type=f32, precision=lax.Precision.HIGHEST
        )

        descs = []
        for d in range(N_DEV):
            descs.append(
                pltpu.make_async_remote_copy(
                    src_ref=send_ref.at[pl.ds(d * BLK, BLK)],
                    dst_ref=recv_ref.at[pl.ds(me * BLK, BLK)],
                    send_sem=send_sems.at[d],
                    recv_sem=recv_sems.at[me],
                    device_id=(d,),
                    device_id_type=pl.DeviceIdType.MESH,
                )
            )
        for d in range(N_DEV):
            pl.when(d != me)(descs[d].start)

        recv_ref[pl.ds(me * BLK, BLK)] = send_ref[pl.ds(me * BLK, BLK)]

        for s in range(N_DEV):
            wdesc = pltpu.make_async_remote_copy(
                src_ref=send_ref.at[pl.ds(s * BLK, BLK)],
                dst_ref=recv_ref.at[pl.ds(s * BLK, BLK)],
                send_sem=send_sems.at[s],
                recv_sem=recv_sems.at[s],
                device_id=(s,),
                device_id_type=pl.DeviceIdType.MESH,
            )
            pl.when(s != me)(wdesc.wait_recv)

        recv_val = recv_ref[:, :]
        Psel = (_iota((N_DEV, FLAT), 1) == _iota((N_DEV, FLAT), 0) * BLK).astype(f32)
        cin_col = jnp.dot(Psel, recv_val, preferred_element_type=f32, precision=lax.Precision.HIGHEST)[:, 0:1]
        eye32 = (_iota((N_DEV, N_DEV), 0) == _iota((N_DEV, N_DEV), 1)).astype(f32)
        cin_row = jnp.dot(
            jnp.ones((1, N_DEV), f32), eye32 * cin_col, preferred_element_type=f32, precision=lax.Precision.HIGHEST
        )
        TU = (_iota((N_DEV, N_DEV), 0) < _iota((N_DEV, N_DEV), 1)).astype(f32)
        cumexcl_row = jnp.dot(cin_row, TU, preferred_element_type=f32, precision=lax.Precision.HIGHEST)

        i_flat = _iota((1, FLAT), 1)
        s_row = i_flat >> 5
        j_row = (i_flat & (BLK - 1)) - 1
        onehot_sT = (_iota((N_DEV, FLAT), 0) == s_row).astype(f32)
        off_f = jnp.dot(cumexcl_row, onehot_sT, preferred_element_type=f32, precision=lax.Precision.HIGHEST)
        cin_f = jnp.dot(cin_row, onehot_sT, preferred_element_type=f32, precision=lax.Precision.HIGHEST)
        j_f = j_row.astype(f32)
        valid = (j_f >= 0.0) & (j_f < cin_f)
        pos_f = off_f + j_f
        k_col = _iota((n_local, 1), 0).astype(f32)
        G = ((jnp.abs(k_col - pos_f) < 0.5) & valid).astype(f32)
        out_ref[:, :] = jnp.dot(G, recv_val, preferred_element_type=f32, precision=lax.Precision.HIGHEST)

        for d in range(N_DEV):
            pl.when(d != me)(descs[d].wait_send)

    return pl.pallas_call(
        body,
        out_shape=jax.ShapeDtypeStruct((n_local, d_model), f32),
        in_specs=[
            pl.BlockSpec(memory_space=pltpu.VMEM),
            pl.BlockSpec(memory_space=pltpu.VMEM),
        ],
        out_specs=pl.BlockSpec(memory_space=pltpu.VMEM),
        scratch_shapes=[
            pltpu.VMEM((FLAT, d_model), f32),
            pltpu.VMEM((FLAT, d_model), f32),
            pltpu.SemaphoreType.DMA((N_DEV,)),
            pltpu.SemaphoreType.DMA((N_DEV,)),
        ],
    )(x, dest_row)
